# Initial kernel scaffold; baseline (speedup 1.0000x reference)
import jax
import jax.numpy as jnp
from jax.experimental import pallas as pl


def kernel(logits, gold, seq_len, W_trans):
    raise NotImplementedError("write your pallas kernel here")


# speedup vs baseline: 14.9339x; 14.9339x over previous
import functools

import jax
import jax.numpy as jnp
from jax.experimental import pallas as pl
from jax.experimental.pallas import tpu as pltpu

B, T, K = 16, 512, 64
UNROLL = 4
NGROUPS = T // UNROLL


def _tc_body(logits_t_ref, gold3_ref, seq3_ref, seq_col_ref, w_ref,
             wt_ref, out_ref, elog_ref):
    lt = logits_t_ref[...]
    gold3 = gold3_ref[...]
    seq3 = seq3_ref[...]
    seq_col = seq_col_ref[...]

    kio = jax.lax.broadcasted_iota(jnp.int32, (T, B, K), 2)
    onehot = gold3 == kio
    tio = jax.lax.broadcasted_iota(jnp.int32, (T, B, K), 0)
    valid = tio < seq3
    first = jnp.sum(jnp.where(onehot & valid, lt, 0.0))

    oh1 = (gold3[: T - 1] == kio[: T - 1]).astype(jnp.float32)
    oh2 = (gold3[1:] == kio[: T - 1]).astype(jnp.float32)
    rows = jax.lax.dot_general(
        oh1.reshape((T - 1) * B, K), w_ref[...],
        (((1,), (0,)), ((), ())),
        preferred_element_type=jnp.float32,
    ).reshape(T - 1, B, K)
    valid2 = tio[: T - 1] < seq3 - 1
    second = jnp.sum(jnp.where(valid2, rows * oh2, 0.0))

    elog_ref[...] = jnp.exp(lt)
    ewt = jnp.exp(wt_ref[...])

    alpha0 = lt[0]
    clog = jnp.max(alpha0, axis=1, keepdims=True)
    p = jnp.exp(alpha0 - clog)
    snap, csnap = p, clog

    def group(r, carry):
        p, clog, snap, csnap = carry
        for u in range(UNROLL):
            t = 1 + r * UNROLL + u
            slot = jnp.minimum(t, T - 1)
            el = elog_ref[slot]
            p = jax.lax.dot_general(
                p, ewt, (((1,), (0,)), ((), ())),
                preferred_element_type=jnp.float32) * el
            hit = t == seq_col - 1
            snap = jnp.where(hit, p, snap)
            csnap = jnp.where(hit, clog, csnap)
        m = jnp.max(p, axis=1, keepdims=True)
        p = p * (1.0 / m)
        clog = clog + jnp.log(m)
        return p, clog, snap, csnap

    _, _, snap, csnap = jax.lax.fori_loop(
        0, NGROUPS, group, (p, clog, snap, csnap))
    third = jnp.sum(jnp.log(jnp.sum(snap, axis=1, keepdims=True)) + csnap)

    out_ref[0] = first
    out_ref[1] = second
    out_ref[2] = third


@functools.partial(jax.jit, static_argnames=("interpret",))
def kernel(logits, gold, seq_len, W_trans, interpret=False):
    logits_t = jnp.transpose(logits, (1, 0, 2))
    gold3 = gold.T.reshape(T, B, 1)
    seq3 = seq_len.reshape(1, B, 1)
    seq_col = seq_len.reshape(B, 1)

    parts = pl.pallas_call(
        _tc_body,
        out_shape=jax.ShapeDtypeStruct((3,), jnp.float32),
        in_specs=[
            pl.BlockSpec(memory_space=pltpu.VMEM),
            pl.BlockSpec(memory_space=pltpu.VMEM),
            pl.BlockSpec(memory_space=pltpu.VMEM),
            pl.BlockSpec(memory_space=pltpu.VMEM),
            pl.BlockSpec(memory_space=pltpu.VMEM),
            pl.BlockSpec(memory_space=pltpu.VMEM),
        ],
        out_specs=pl.BlockSpec(memory_space=pltpu.SMEM),
        scratch_shapes=[pltpu.VMEM((T, B, K), jnp.float32)],
        interpret=interpret,
    )(logits_t, gold3, seq3, seq_col, W_trans, W_trans.T)

    first, second, third = parts[0], parts[1], parts[2]
    return -(first + second - third) / jnp.float32(B)
